# baseline (speedup 1.0000x reference)
import jax
import jax.numpy as jnp
from jax.experimental import pallas as pl
from jax.experimental.pallas import tpu as pltpu

_EPS = 1e-3
_P = 3200
_NV = 10000
_NEG = float(jnp.finfo(jnp.float32).min)


def _ln_t(x, g, b):
    m = x.mean(0, keepdims=True)
    v = ((x - m) ** 2).mean(0, keepdims=True)
    return (x - m) / jnp.sqrt(v + _EPS) * g + b


def _seg_scan_max_t(x, seg, reverse):
    f, n = x.shape
    d = 1
    while d < n:
        if not reverse:
            xs = jnp.concatenate([jnp.full((f, d), _NEG, x.dtype), x[:, :-d]], 1)
            ss = jnp.concatenate([jnp.full((1, d), -1, seg.dtype), seg[:, :-d]], 1)
        else:
            xs = jnp.concatenate([x[:, d:], jnp.full((f, d), _NEG, x.dtype)], 1)
            ss = jnp.concatenate([seg[:, d:], jnp.full((1, d), -1, seg.dtype)], 1)
        x = jnp.maximum(x, jnp.where(ss == seg, xs, _NEG))
        d *= 2
    return x


def _init_carry(cvec, cseg):
    @pl.when(pl.program_id(0) == 0)
    def _():
        cvec[...] = jnp.full(cvec.shape, _NEG, jnp.float32)
        cseg[0] = -1
        cseg[1] = -1


def _stage1(pts, a, b, fcl, w0p, w0a, w0b, b0, g0, bt0, rw1, rb1, rw2, rb2):
    dot = lambda w, v: jnp.dot(w, v, preferred_element_type=jnp.float32)
    pre = dot(w0p, pts) + dot(w0a, a)
    if b is not None:
        pre = pre + dot(w0b, b)
    pre = pre + b0
    x = jnp.maximum(_ln_t(pre, g0, bt0), 0.0)
    rel = jnp.maximum(dot(rw1, fcl / 10.0) + rb1, 0.0)
    rel = dot(rw2, rel) + rb2
    return x + rel


def _entry_fwd_kernel(seg_ref, pts_ref, a_ref, fcl_ref,
                      w0p_ref, w0a_ref, b0_ref, g0_ref, bt0_ref,
                      rw1_ref, rb1_ref, rw2_ref, rb2_ref,
                      x_out, fx_out, cvec, cseg):
    _init_carry(cvec, cseg)
    seg = seg_ref[...]
    x = _stage1(pts_ref[...], a_ref[...], None, fcl_ref[...],
                w0p_ref[...], w0a_ref[...], None,
                b0_ref[...], g0_ref[...], bt0_ref[...],
                rw1_ref[...], rb1_ref[...], rw2_ref[...], rb2_ref[...])
    x_out[...] = x
    fx = _seg_scan_max_t(x, seg, reverse=False)
    fx = jnp.where(seg == cseg[0], jnp.maximum(fx, cvec[:, :1]), fx)
    fx_out[...] = fx
    cvec[:, :1] = fx[:, -1:]
    cseg[0] = seg[0, -1]


def _rev_kernel(seg_ref, x_ref, fx_ref,
                w1a_ref, w1b_ref, b1_ref, g1_ref, bt1_ref,
                y_out, by_out, cvec, cseg):
    _init_carry(cvec, cseg)
    seg = seg_ref[...]
    x = x_ref[...]
    bx = _seg_scan_max_t(x, seg, reverse=True)
    bx = jnp.where(seg == cseg[0], jnp.maximum(bx, cvec[:, :1]), bx)
    cvec[:, :1] = bx[:, :1]
    cseg[0] = seg[0, 0]

    c0 = jnp.maximum(fx_ref[...], bx)
    dot = lambda w, v: jnp.dot(w, v, preferred_element_type=jnp.float32)
    pre = dot(w1a_ref[...], x) + dot(w1b_ref[...], c0) + b1_ref[...]
    y = jnp.maximum(_ln_t(pre, g1_ref[...], bt1_ref[...]), 0.0)
    y_out[...] = y

    by = _seg_scan_max_t(y, seg, reverse=True)
    by = jnp.where(seg == cseg[1], jnp.maximum(by, cvec[:, 1:]), by)
    by_out[...] = by
    cvec[:, 1:] = by[:, :1]
    cseg[1] = seg[0, 0]


def _fwd_next_kernel(seg_ref, y_ref, by_ref, pts_ref, fcl_ref,
                     w0p_ref, w0y_ref, w0c_ref, b0_ref, g0_ref, bt0_ref,
                     rw1_ref, rb1_ref, rw2_ref, rb2_ref,
                     c_out, x_out, fx_out, cvec, cseg):
    _init_carry(cvec, cseg)
    seg = seg_ref[...]
    y = y_ref[...]
    fy = _seg_scan_max_t(y, seg, reverse=False)
    fy = jnp.where(seg == cseg[0], jnp.maximum(fy, cvec[:, :1]), fy)
    cvec[:, :1] = fy[:, -1:]
    cseg[0] = seg[0, -1]
    c1 = jnp.maximum(fy, by_ref[...])
    c_out[...] = c1

    x = _stage1(pts_ref[...], y, c1, fcl_ref[...],
                w0p_ref[...], w0y_ref[...], w0c_ref[...],
                b0_ref[...], g0_ref[...], bt0_ref[...],
                rw1_ref[...], rb1_ref[...], rw2_ref[...], rb2_ref[...])
    x_out[...] = x
    fx = _seg_scan_max_t(x, seg, reverse=False)
    fx = jnp.where(seg == cseg[1], jnp.maximum(fx, cvec[:, 1:]), fx)
    fx_out[...] = fx
    cvec[:, 1:] = fx[:, -1:]
    cseg[1] = seg[0, -1]


def _fwd_final_kernel(seg_ref, y_ref, by_ref, c_out, cvec, cseg):
    _init_carry(cvec, cseg)
    seg = seg_ref[...]
    fy = _seg_scan_max_t(y_ref[...], seg, reverse=False)
    fy = jnp.where(seg == cseg[0], jnp.maximum(fy, cvec[:, :1]), fy)
    cvec[:, :1] = fy[:, -1:]
    cseg[0] = seg[0, -1]
    c_out[...] = jnp.maximum(fy, by_ref[...])


def _full(shape):
    return pl.BlockSpec(shape, lambda c: tuple(0 for _ in shape))


def _chunk(height, rev, nc):
    if rev:
        return pl.BlockSpec((height, _P), lambda c: (0, nc - 1 - c))
    return pl.BlockSpec((height, _P), lambda c: (0, c))


def _scratch():
    return [pltpu.VMEM((32, 2), jnp.float32), pltpu.SMEM((2,), jnp.int32)]


def _cparams():
    return pltpu.CompilerParams(dimension_semantics=("arbitrary",))


def kernel(points, features, coors, f_cluster, params):
    n = points.shape[0]
    nc = n // _P
    coors = coors.astype(jnp.int32)

    is_new = jnp.concatenate(
        [jnp.ones((1,), jnp.int32), (coors[1:] != coors[:-1]).astype(jnp.int32)])
    unq_inv = (jnp.cumsum(is_new) - 1).astype(jnp.int32)
    n_act = unq_inv[-1] + 1
    unq = jnp.full((_NV,), coors[0], coors.dtype).at[unq_inv].set(coors)
    seg_starts = jnp.searchsorted(coors, unq).astype(jnp.int32)
    seg_t = unq_inv.reshape(1, n)

    pts_t = points.T
    fcl_t = f_cluster.T
    f32 = jnp.float32
    out32 = jax.ShapeDtypeStruct((32, n), f32)
    col = lambda v: v.reshape(-1, 1)

    def rel_args(p):
        return (p['rel_W1'].T, col(p['rel_b1']), p['rel_W2'].T, col(p['rel_b2']))

    def ln0_args(p):
        return (col(p['b0']), col(p['g0']), col(p['bt0']))

    p0 = params[0]
    x, fx = pl.pallas_call(
        _entry_fwd_kernel,
        grid=(nc,),
        in_specs=[_chunk(1, False, nc), _chunk(4, False, nc),
                  _chunk(12, False, nc), _chunk(3, False, nc),
                  _full((32, 4)), _full((32, 12)),
                  _full((32, 1)), _full((32, 1)), _full((32, 1)),
                  _full((16, 3)), _full((16, 1)), _full((32, 16)),
                  _full((32, 1))],
        out_specs=[_chunk(32, False, nc), _chunk(32, False, nc)],
        out_shape=[out32, out32],
        scratch_shapes=_scratch(),
        compiler_params=_cparams(),
    )(seg_t, pts_t, features.T, fcl_t,
      p0['W0'][:4].T, p0['W0'][4:16].T, *ln0_args(p0), *rel_args(p0))

    def rev(p, x, fx):
        return pl.pallas_call(
            _rev_kernel,
            grid=(nc,),
            in_specs=[_chunk(1, True, nc), _chunk(32, True, nc),
                      _chunk(32, True, nc),
                      _full((32, 32)), _full((32, 32)),
                      _full((32, 1)), _full((32, 1)), _full((32, 1))],
            out_specs=[_chunk(32, True, nc), _chunk(32, True, nc)],
            out_shape=[out32, out32],
            scratch_shapes=_scratch(),
            compiler_params=_cparams(),
        )(seg_t, x, fx, p['W1'][:32].T, p['W1'][32:].T,
          col(p['b1']), col(p['g1']), col(p['bt1']))

    def fwd_next(pn, y, by):
        return pl.pallas_call(
            _fwd_next_kernel,
            grid=(nc,),
            in_specs=[_chunk(1, False, nc), _chunk(32, False, nc),
                      _chunk(32, False, nc), _chunk(4, False, nc),
                      _chunk(3, False, nc),
                      _full((32, 4)), _full((32, 32)), _full((32, 32)),
                      _full((32, 1)), _full((32, 1)), _full((32, 1)),
                      _full((16, 3)), _full((16, 1)), _full((32, 16)),
                      _full((32, 1))],
            out_specs=[_chunk(32, False, nc)] * 3,
            out_shape=[out32, out32, out32],
            scratch_shapes=_scratch(),
            compiler_params=_cparams(),
        )(seg_t, y, by, pts_t, fcl_t,
          pn['W0'][:4].T, pn['W0'][4:36].T, pn['W0'][36:].T,
          *ln0_args(pn), *rel_args(pn))

    pooled_pts = []
    y, by = rev(params[0], x, fx)
    c1, x, fx = fwd_next(params[1], y, by)
    pooled_pts.append(c1)
    y, by = rev(params[1], x, fx)
    c1, x, fx = fwd_next(params[2], y, by)
    pooled_pts.append(c1)
    y, by = rev(params[2], x, fx)
    c1 = pl.pallas_call(
        _fwd_final_kernel,
        grid=(nc,),
        in_specs=[_chunk(1, False, nc), _chunk(32, False, nc),
                  _chunk(32, False, nc)],
        out_specs=[_chunk(32, False, nc)],
        out_shape=[out32],
        scratch_shapes=_scratch(),
        compiler_params=_cparams(),
    )(seg_t, y, by)[0]
    pooled_pts.append(c1)

    out_feats = jnp.concatenate([y, c1], axis=0).T
    pooled = jnp.concatenate(
        [jnp.take(c, seg_starts, axis=1) for c in pooled_pts], axis=0).T
    mask = (jnp.arange(_NV) < n_act)[:, None]
    final_cluster_feats = jnp.where(mask, pooled, -jnp.inf)
    return (out_feats, final_cluster_feats, unq)
